# baseline (speedup 1.0000x reference)
import functools

import jax
import jax.numpy as jnp
from jax import lax
from jax.experimental import pallas as pl
from jax.experimental.pallas import tpu as pltpu
from jax.experimental.pallas import tpu_sc as plsc

B = 2048
E = 256
V = 512

_NC = 2
_NS = 16
_NW = _NC * _NS
_BPW = B // _NW

def _tc_body(x_ref, w_ref, idx_ref, wt_ref):
    w = w_ref[...]
    wnorm = jnp.sum(w * w, axis=0, keepdims=True)
    scores = lax.dot_general(
        x_ref[...], w,
        (((1,), (0,)), ((), ())),
        preferred_element_type=jnp.float32,
        precision=lax.Precision.HIGHEST,
    )
    d = wnorm - 2.0 * scores
    idx_ref[...] = jnp.argmin(d, axis=1).astype(jnp.int32)
    wt_ref[...] = w.T


_tc_call = pl.pallas_call(
    _tc_body,
    out_shape=(
        jax.ShapeDtypeStruct((B,), jnp.int32),
        jax.ShapeDtypeStruct((V, E), jnp.float32),
    ),
)


@functools.cache
def _sc_gather_call():
    mesh = plsc.VectorSubcoreMesh(core_axis_name="c", subcore_axis_name="s")

    @functools.partial(
        pl.kernel,
        mesh=mesh,
        compiler_params=pltpu.CompilerParams(use_tc_tiling_on_sc=False),
        out_type=jax.ShapeDtypeStruct((B, E), jnp.float32),
        scratch_types=[
            pltpu.VMEM((_BPW,), jnp.int32),
            pltpu.VMEM((_BPW, E), jnp.float32),
            pltpu.SemaphoreType.DMA,
        ],
    )
    def _sc_gather(table_hbm, idx_hbm, out_hbm, idx_v, rows_v, sem):
        wid = lax.axis_index("s") * _NC + lax.axis_index("c")
        base = wid * _BPW
        pltpu.sync_copy(idx_hbm.at[pl.ds(base, _BPW)], idx_v)
        pltpu.async_copy(table_hbm.at[idx_v], rows_v, sem).wait()
        pltpu.sync_copy(rows_v, out_hbm.at[pl.ds(base, _BPW)])

    return _sc_gather


def kernel(x, weight):
    idx, table = _tc_call(x, weight)
    return _sc_gather_call()(table, idx)


# speedup vs baseline: 1.1762x; 1.1762x over previous
import functools

import jax
import jax.numpy as jnp
from jax import lax
from jax.experimental import pallas as pl
from jax.experimental.pallas import tpu as pltpu
from jax.experimental.pallas import tpu_sc as plsc

B = 2048
E = 256
V = 512

_NC = 2
_NS = 16
_NW = _NC * _NS
_BPW = B // _NW

def _tc_body(x_ref, w_ref, idx_ref, wt_ref):
    w = w_ref[...]
    wnorm = jnp.sum(w * w, axis=0, keepdims=True)
    scores = lax.dot_general(
        x_ref[...], w,
        (((1,), (0,)), ((), ())),
        preferred_element_type=jnp.float32,
        precision=lax.Precision.HIGHEST,
    )
    d = wnorm - 2.0 * scores
    idx_ref[...] = jnp.argmin(d, axis=1).astype(jnp.int32)
    wt_ref[...] = w.T


_GRID = 2
_BR = B // _GRID

_tc_call = pl.pallas_call(
    _tc_body,
    grid=(_GRID,),
    in_specs=[
        pl.BlockSpec((_BR, E), lambda i: (i, 0)),
        pl.BlockSpec((E, V), lambda i: (0, 0)),
    ],
    out_specs=(
        pl.BlockSpec((_BR,), lambda i: (i,)),
        pl.BlockSpec((V, E), lambda i: (0, 0)),
    ),
    out_shape=(
        jax.ShapeDtypeStruct((B,), jnp.int32),
        jax.ShapeDtypeStruct((V, E), jnp.float32),
    ),
)


@functools.cache
def _sc_gather_call():
    mesh = plsc.VectorSubcoreMesh(core_axis_name="c", subcore_axis_name="s")

    @functools.partial(
        pl.kernel,
        mesh=mesh,
        out_type=jax.ShapeDtypeStruct((B, E), jnp.float32),
        scratch_types=[
            pltpu.VMEM((_BPW,), jnp.int32),
            pltpu.VMEM((_BPW, E), jnp.float32),
            pltpu.SemaphoreType.DMA,
        ],
    )
    def _sc_gather(table_hbm, idx_hbm, out_hbm, idx_v, rows_v, sem):
        wid = lax.axis_index("s") * _NC + lax.axis_index("c")
        base = wid * _BPW
        pltpu.sync_copy(idx_hbm.at[pl.ds(base, _BPW)], idx_v)
        pltpu.async_copy(table_hbm.at[idx_v], rows_v, sem).wait()
        pltpu.sync_copy(rows_v, out_hbm.at[pl.ds(base, _BPW)])

    return _sc_gather


def kernel(x, weight):
    idx, table = _tc_call(x, weight)
    return _sc_gather_call()(table, idx)
